# baseline (device time: 27607 ns/iter reference)
import jax
import jax.numpy as jnp
from jax import lax
from jax.experimental import pallas as pl
from jax.experimental.pallas import tpu as pltpu

N_DEV = 4
E_PER = 4
N_TOK = 1024
N_EXP = 16
D_IN = 256
D_OUT = 512
CHUNK = N_TOK // N_DEV


def kernel(x, router_W, route_idx, expert_W, shared_W):
    def body(x_ref, rw_ref, idx_ref, ew_ref, sw_ref, out_ref,
             sb_ref, red_ref, ag_send_ref, rs_buf, ag_buf,
             rs_send_sems, rs_recv_sems, ag_send_sems, ag_recv_sems):
        my = lax.axis_index("i")

        barrier_sem = pltpu.get_barrier_semaphore()
        for s in range(1, N_DEV):
            peer = lax.rem(my + s, N_DEV)
            pl.semaphore_signal(
                barrier_sem, inc=1,
                device_id=(peer,), device_id_type=pl.DeviceIdType.MESH,
            )
        pl.semaphore_wait(barrier_sem, N_DEV - 1)

        x32 = x_ref[:, :]
        scores = jnp.dot(x32, rw_ref[:, :], preferred_element_type=jnp.float32)
        s_max = jnp.max(scores, axis=-1, keepdims=True)
        e_s = jnp.exp(scores - s_max)
        probs = e_s / jnp.sum(e_s, axis=-1, keepdims=True)
        idx2 = idx_ref[:, :]
        eiota = lax.broadcasted_iota(jnp.int32, (N_TOK, N_EXP), 1)
        p_routed = jnp.sum(
            jnp.where(eiota == idx2, probs, 0.0), axis=-1, keepdims=True
        )

        xbf = x32.astype(jnp.bfloat16)
        blocks = []
        for j in range(E_PER):
            e_glob = my * E_PER + j
            scale = jnp.where(idx2 == e_glob, p_routed, 0.0).astype(jnp.bfloat16)
            blocks.append(xbf * scale)
        x_stack = jnp.concatenate(blocks, axis=1)
        w_stack = jnp.concatenate(
            [ew_ref[j].astype(jnp.bfloat16) for j in range(E_PER)], axis=0
        )

        rs_sends = []
        for c in range(N_DEV):
            part = jnp.dot(
                x_stack[c * CHUNK:(c + 1) * CHUNK, :], w_stack,
                preferred_element_type=jnp.float32,
            )

            @pl.when(my == c)
            def _(part=part):
                red_ref[:, :] = part

            @pl.when(my != c)
            def _(c=c, part=part):
                sb_ref[c] = part.astype(jnp.bfloat16)
                rdma = pltpu.make_async_remote_copy(
                    src_ref=sb_ref.at[c],
                    dst_ref=rs_buf.at[my],
                    send_sem=rs_send_sems.at[c],
                    recv_sem=rs_recv_sems.at[my],
                    device_id=(c,),
                    device_id_type=pl.DeviceIdType.MESH,
                )
                rdma.start()

        shared = jnp.dot(
            xbf, sw_ref[:, :].astype(jnp.bfloat16),
            preferred_element_type=jnp.float32,
        )
        out_ref[:, :] = shared

        for k in range(N_DEV):
            @pl.when(my != k)
            def _(k=k):
                recv = pltpu.make_async_remote_copy(
                    src_ref=sb_ref.at[k],
                    dst_ref=rs_buf.at[k],
                    send_sem=rs_send_sems.at[k],
                    recv_sem=rs_recv_sems.at[k],
                    device_id=(k,),
                    device_id_type=pl.DeviceIdType.MESH,
                )
                recv.wait_recv()
                red_ref[:, :] += rs_buf[k].astype(jnp.float32)

        out_ref[pl.ds(my * CHUNK, CHUNK), :] += red_ref[:, :]
        ag_send_ref[:, :] = red_ref[:, :].astype(jnp.bfloat16)

        ag_sends = []
        for c in range(N_DEV):
            @pl.when(my != c)
            def _(c=c):
                rdma = pltpu.make_async_remote_copy(
                    src_ref=ag_send_ref,
                    dst_ref=ag_buf.at[my],
                    send_sem=ag_send_sems.at[c],
                    recv_sem=ag_recv_sems.at[my],
                    device_id=(c,),
                    device_id_type=pl.DeviceIdType.MESH,
                )
                rdma.start()

        for k in range(N_DEV):
            @pl.when(my != k)
            def _(k=k):
                recv = pltpu.make_async_remote_copy(
                    src_ref=ag_send_ref,
                    dst_ref=ag_buf.at[k],
                    send_sem=ag_send_sems.at[k],
                    recv_sem=ag_recv_sems.at[k],
                    device_id=(k,),
                    device_id_type=pl.DeviceIdType.MESH,
                )
                recv.wait_recv()
                out_ref[k * CHUNK:(k + 1) * CHUNK, :] += ag_buf[k].astype(
                    jnp.float32
                )

        for c in range(N_DEV):
            @pl.when(my != c)
            def _(c=c):
                send = pltpu.make_async_remote_copy(
                    src_ref=sb_ref.at[c],
                    dst_ref=rs_buf.at[my],
                    send_sem=rs_send_sems.at[c],
                    recv_sem=rs_recv_sems.at[my],
                    device_id=(c,),
                    device_id_type=pl.DeviceIdType.MESH,
                )
                send.wait_send()
                send2 = pltpu.make_async_remote_copy(
                    src_ref=ag_send_ref,
                    dst_ref=ag_buf.at[my],
                    send_sem=ag_send_sems.at[c],
                    recv_sem=ag_recv_sems.at[my],
                    device_id=(c,),
                    device_id_type=pl.DeviceIdType.MESH,
                )
                send2.wait_send()

    return pl.pallas_call(
        body,
        out_shape=jax.ShapeDtypeStruct((N_TOK, D_OUT), jnp.float32),
        in_specs=[pl.BlockSpec(memory_space=pltpu.VMEM)] * 5,
        out_specs=pl.BlockSpec(memory_space=pltpu.VMEM),
        scratch_shapes=[
            pltpu.VMEM((N_DEV, CHUNK, D_OUT), jnp.bfloat16),
            pltpu.VMEM((CHUNK, D_OUT), jnp.float32),
            pltpu.VMEM((CHUNK, D_OUT), jnp.bfloat16),
            pltpu.VMEM((N_DEV, CHUNK, D_OUT), jnp.bfloat16),
            pltpu.VMEM((N_DEV, CHUNK, D_OUT), jnp.bfloat16),
            pltpu.SemaphoreType.DMA((N_DEV,)),
            pltpu.SemaphoreType.DMA((N_DEV,)),
            pltpu.SemaphoreType.DMA((N_DEV,)),
            pltpu.SemaphoreType.DMA((N_DEV,)),
        ],
        compiler_params=pltpu.CompilerParams(collective_id=0),
    )(x, router_W, route_idx, expert_W, shared_W)


# device time: 25041 ns/iter; 1.1025x vs baseline; 1.1025x over previous
import jax
import jax.numpy as jnp
from jax import lax
from jax.experimental import pallas as pl
from jax.experimental.pallas import tpu as pltpu

N_DEV = 4
E_PER = 4
N_TOK = 1024
N_EXP = 16
D_IN = 256
D_OUT = 512
CHUNK = N_TOK // N_DEV
HALVES = 2
SUB = CHUNK // HALVES


def kernel(x, router_W, route_idx, expert_W, shared_W):
    def body(x_ref, rw_ref, idx_ref, ew_ref, sw_ref, out_ref,
             sb_ref, red_ref, ag_send_ref, rs_buf, ag_buf,
             rs_send_sems, rs_recv_sems, ag_send_sems, ag_recv_sems):
        my = lax.axis_index("i")

        barrier_sem = pltpu.get_barrier_semaphore()
        for s in range(1, N_DEV):
            peer = lax.rem(my + s, N_DEV)
            pl.semaphore_signal(
                barrier_sem, inc=1,
                device_id=(peer,), device_id_type=pl.DeviceIdType.MESH,
            )
        pl.semaphore_wait(barrier_sem, N_DEV - 1)

        x32 = x_ref[:, :]
        scores = jnp.dot(x32, rw_ref[:, :], preferred_element_type=jnp.float32)
        s_max = jnp.max(scores, axis=-1, keepdims=True)
        e_s = jnp.exp(scores - s_max)
        probs = e_s / jnp.sum(e_s, axis=-1, keepdims=True)
        idx2 = idx_ref[:, :]
        eiota = lax.broadcasted_iota(jnp.int32, (N_TOK, N_EXP), 1)
        p_routed = jnp.sum(
            jnp.where(eiota == idx2, probs, 0.0), axis=-1, keepdims=True
        )

        xbf = x32.astype(jnp.bfloat16)
        w_stack = jnp.concatenate(
            [ew_ref[j].astype(jnp.bfloat16) for j in range(E_PER)], axis=0
        )
        scales = []
        for j in range(E_PER):
            e_glob = my * E_PER + j
            scales.append(
                jnp.where(idx2 == e_glob, p_routed, 0.0).astype(jnp.bfloat16)
            )

        def partial_rows(lo, n):
            xs = jnp.concatenate(
                [xbf[lo:lo + n, :] * s[lo:lo + n, :] for s in scales], axis=1
            )
            return jnp.dot(xs, w_stack, preferred_element_type=jnp.float32)

        rs_sends = []
        for h in range(HALVES):
            for c in range(N_DEV):
                lo = c * CHUNK + h * SUB
                part = partial_rows(lo, SUB)

                @pl.when(my == c)
                def _(h=h, part=part):
                    red_ref[h * SUB:(h + 1) * SUB, :] = part

                @pl.when(my != c)
                def _(h=h, c=c, part=part):
                    sb_ref[h, c] = part.astype(jnp.bfloat16)
                    rdma = pltpu.make_async_remote_copy(
                        src_ref=sb_ref.at[h, c],
                        dst_ref=rs_buf.at[h, my],
                        send_sem=rs_send_sems.at[h, c],
                        recv_sem=rs_recv_sems.at[h, my],
                        device_id=(c,),
                        device_id_type=pl.DeviceIdType.MESH,
                    )
                    rdma.start()

        shared = jnp.dot(
            xbf, sw_ref[:, :].astype(jnp.bfloat16),
            preferred_element_type=jnp.float32,
        )
        out_ref[:, :] = shared

        for h in range(HALVES):
            for k in range(N_DEV):
                @pl.when(my != k)
                def _(h=h, k=k):
                    recv = pltpu.make_async_remote_copy(
                        src_ref=sb_ref.at[h, k],
                        dst_ref=rs_buf.at[h, k],
                        send_sem=rs_send_sems.at[h, k],
                        recv_sem=rs_recv_sems.at[h, k],
                        device_id=(k,),
                        device_id_type=pl.DeviceIdType.MESH,
                    )
                    recv.wait_recv()
                    red_ref[h * SUB:(h + 1) * SUB, :] += rs_buf[h, k].astype(
                        jnp.float32
                    )
            ag_send_ref[h] = red_ref[h * SUB:(h + 1) * SUB, :].astype(
                jnp.bfloat16
            )
            for c in range(N_DEV):
                @pl.when(my != c)
                def _(h=h, c=c):
                    rdma = pltpu.make_async_remote_copy(
                        src_ref=ag_send_ref.at[h],
                        dst_ref=ag_buf.at[h, my],
                        send_sem=ag_send_sems.at[h, c],
                        recv_sem=ag_recv_sems.at[h, my],
                        device_id=(c,),
                        device_id_type=pl.DeviceIdType.MESH,
                    )
                    rdma.start()

        out_ref[pl.ds(my * CHUNK, CHUNK), :] += red_ref[:, :]

        for h in range(HALVES):
            for k in range(N_DEV):
                @pl.when(my != k)
                def _(h=h, k=k):
                    recv = pltpu.make_async_remote_copy(
                        src_ref=ag_send_ref.at[h],
                        dst_ref=ag_buf.at[h, k],
                        send_sem=ag_send_sems.at[h, k],
                        recv_sem=ag_recv_sems.at[h, k],
                        device_id=(k,),
                        device_id_type=pl.DeviceIdType.MESH,
                    )
                    recv.wait_recv()
                    lo = k * CHUNK + h * SUB
                    out_ref[lo:lo + SUB, :] += ag_buf[h, k].astype(jnp.float32)

        for h in range(HALVES):
            for c in range(N_DEV):
                @pl.when(my != c)
                def _(h=h, c=c):
                    send = pltpu.make_async_remote_copy(
                        src_ref=sb_ref.at[h, c],
                        dst_ref=rs_buf.at[h, my],
                        send_sem=rs_send_sems.at[h, c],
                        recv_sem=rs_recv_sems.at[h, my],
                        device_id=(c,),
                        device_id_type=pl.DeviceIdType.MESH,
                    )
                    send.wait_send()
                    send2 = pltpu.make_async_remote_copy(
                        src_ref=ag_send_ref.at[h],
                        dst_ref=ag_buf.at[h, my],
                        send_sem=ag_send_sems.at[h, c],
                        recv_sem=ag_recv_sems.at[h, my],
                        device_id=(c,),
                        device_id_type=pl.DeviceIdType.MESH,
                    )
                    send2.wait_send()

    return pl.pallas_call(
        body,
        out_shape=jax.ShapeDtypeStruct((N_TOK, D_OUT), jnp.float32),
        in_specs=[pl.BlockSpec(memory_space=pltpu.VMEM)] * 5,
        out_specs=pl.BlockSpec(memory_space=pltpu.VMEM),
        scratch_shapes=[
            pltpu.VMEM((HALVES, N_DEV, SUB, D_OUT), jnp.bfloat16),
            pltpu.VMEM((CHUNK, D_OUT), jnp.float32),
            pltpu.VMEM((HALVES, SUB, D_OUT), jnp.bfloat16),
            pltpu.VMEM((HALVES, N_DEV, SUB, D_OUT), jnp.bfloat16),
            pltpu.VMEM((HALVES, N_DEV, SUB, D_OUT), jnp.bfloat16),
            pltpu.SemaphoreType.DMA((HALVES, N_DEV)),
            pltpu.SemaphoreType.DMA((HALVES, N_DEV)),
            pltpu.SemaphoreType.DMA((HALVES, N_DEV)),
            pltpu.SemaphoreType.DMA((HALVES, N_DEV)),
        ],
        compiler_params=pltpu.CompilerParams(collective_id=0),
    )(x, router_W, route_idx, expert_W, shared_W)


# device time: 24799 ns/iter; 1.1132x vs baseline; 1.0098x over previous
import jax
import jax.numpy as jnp
from jax import lax
from jax.experimental import pallas as pl
from jax.experimental.pallas import tpu as pltpu

N_DEV = 4
E_PER = 4
N_TOK = 1024
N_EXP = 16
D_IN = 256
D_OUT = 512
CHUNK = N_TOK // N_DEV
HALVES = 2
SUB = CHUNK // HALVES


def kernel(x, router_W, route_idx, expert_W, shared_W):
    def body(x_ref, rw_ref, idx_ref, ew_ref, sw_ref, out_ref,
             sb_ref, red_ref, ag_send_ref, rs_buf, ag_buf,
             rs_send_sems, rs_recv_sems, ag_send_sems, ag_recv_sems):
        my = lax.axis_index("i")

        barrier_sem = pltpu.get_barrier_semaphore()
        for s in range(1, N_DEV):
            peer = lax.rem(my + s, N_DEV)
            pl.semaphore_signal(
                barrier_sem, inc=1,
                device_id=(peer,), device_id_type=pl.DeviceIdType.MESH,
            )
        pl.semaphore_wait(barrier_sem, N_DEV - 1)

        x32 = x_ref[:, :]
        scores = jnp.dot(x32, rw_ref[:, :], preferred_element_type=jnp.float32)
        s_max = jnp.max(scores, axis=-1, keepdims=True)
        e_s = jnp.exp(scores - s_max)
        probs = e_s / jnp.sum(e_s, axis=-1, keepdims=True)
        idx2 = idx_ref[:, :]
        eiota = lax.broadcasted_iota(jnp.int32, (N_TOK, N_EXP), 1)
        p_routed = jnp.sum(
            jnp.where(eiota == idx2, probs, 0.0), axis=-1, keepdims=True
        )

        xbf = x32.astype(jnp.bfloat16)
        xq = (x32 * 0.25).astype(jnp.bfloat16)
        w_stack = jnp.concatenate(
            [sw_ref[:, :].astype(jnp.bfloat16)]
            + [ew_ref[j].astype(jnp.bfloat16) for j in range(E_PER)],
            axis=0,
        )
        scales = []
        for j in range(E_PER):
            e_glob = my * E_PER + j
            scales.append(
                jnp.where(idx2 == e_glob, p_routed, 0.0).astype(jnp.bfloat16)
            )

        def partial_rows(lo, n):
            xs = jnp.concatenate(
                [xq[lo:lo + n, :]]
                + [xbf[lo:lo + n, :] * s[lo:lo + n, :] for s in scales],
                axis=1,
            )
            return jnp.dot(xs, w_stack, preferred_element_type=jnp.float32)

        for h in range(HALVES):
            for c in range(N_DEV):
                lo = c * CHUNK + h * SUB
                part = partial_rows(lo, SUB)

                @pl.when(my == c)
                def _(h=h, part=part):
                    red_ref[h * SUB:(h + 1) * SUB, :] = part

                @pl.when(my != c)
                def _(h=h, c=c, part=part):
                    sb_ref[h, c] = part.astype(jnp.bfloat16)
                    rdma = pltpu.make_async_remote_copy(
                        src_ref=sb_ref.at[h, c],
                        dst_ref=rs_buf.at[h, my],
                        send_sem=rs_send_sems.at[h, c],
                        recv_sem=rs_recv_sems.at[h, my],
                        device_id=(c,),
                        device_id_type=pl.DeviceIdType.MESH,
                    )
                    rdma.start()

        for h in range(HALVES):
            for k in range(N_DEV):
                @pl.when(my != k)
                def _(h=h, k=k):
                    recv = pltpu.make_async_remote_copy(
                        src_ref=sb_ref.at[h, k],
                        dst_ref=rs_buf.at[h, k],
                        send_sem=rs_send_sems.at[h, k],
                        recv_sem=rs_recv_sems.at[h, k],
                        device_id=(k,),
                        device_id_type=pl.DeviceIdType.MESH,
                    )
                    recv.wait_recv()
                    red_ref[h * SUB:(h + 1) * SUB, :] += rs_buf[h, k].astype(
                        jnp.float32
                    )
            ag_send_ref[h] = red_ref[h * SUB:(h + 1) * SUB, :].astype(
                jnp.bfloat16
            )
            for c in range(N_DEV):
                @pl.when(my != c)
                def _(h=h, c=c):
                    rdma = pltpu.make_async_remote_copy(
                        src_ref=ag_send_ref.at[h],
                        dst_ref=ag_buf.at[h, my],
                        send_sem=ag_send_sems.at[h, c],
                        recv_sem=ag_recv_sems.at[h, my],
                        device_id=(c,),
                        device_id_type=pl.DeviceIdType.MESH,
                    )
                    rdma.start()

        out_ref[pl.ds(my * CHUNK, CHUNK), :] = red_ref[:, :]

        for h in range(HALVES):
            for k in range(N_DEV):
                @pl.when(my != k)
                def _(h=h, k=k):
                    recv = pltpu.make_async_remote_copy(
                        src_ref=ag_send_ref.at[h],
                        dst_ref=ag_buf.at[h, k],
                        send_sem=ag_send_sems.at[h, k],
                        recv_sem=ag_recv_sems.at[h, k],
                        device_id=(k,),
                        device_id_type=pl.DeviceIdType.MESH,
                    )
                    recv.wait_recv()
                    lo = k * CHUNK + h * SUB
                    out_ref[lo:lo + SUB, :] = ag_buf[h, k].astype(jnp.float32)

        for h in range(HALVES):
            for c in range(N_DEV):
                @pl.when(my != c)
                def _(h=h, c=c):
                    send = pltpu.make_async_remote_copy(
                        src_ref=sb_ref.at[h, c],
                        dst_ref=rs_buf.at[h, my],
                        send_sem=rs_send_sems.at[h, c],
                        recv_sem=rs_recv_sems.at[h, my],
                        device_id=(c,),
                        device_id_type=pl.DeviceIdType.MESH,
                    )
                    send.wait_send()
                    send2 = pltpu.make_async_remote_copy(
                        src_ref=ag_send_ref.at[h],
                        dst_ref=ag_buf.at[h, my],
                        send_sem=ag_send_sems.at[h, c],
                        recv_sem=ag_recv_sems.at[h, my],
                        device_id=(c,),
                        device_id_type=pl.DeviceIdType.MESH,
                    )
                    send2.wait_send()

    return pl.pallas_call(
        body,
        out_shape=jax.ShapeDtypeStruct((N_TOK, D_OUT), jnp.float32),
        in_specs=[pl.BlockSpec(memory_space=pltpu.VMEM)] * 5,
        out_specs=pl.BlockSpec(memory_space=pltpu.VMEM),
        scratch_shapes=[
            pltpu.VMEM((HALVES, N_DEV, SUB, D_OUT), jnp.bfloat16),
            pltpu.VMEM((CHUNK, D_OUT), jnp.float32),
            pltpu.VMEM((HALVES, SUB, D_OUT), jnp.bfloat16),
            pltpu.VMEM((HALVES, N_DEV, SUB, D_OUT), jnp.bfloat16),
            pltpu.VMEM((HALVES, N_DEV, SUB, D_OUT), jnp.bfloat16),
            pltpu.SemaphoreType.DMA((HALVES, N_DEV)),
            pltpu.SemaphoreType.DMA((HALVES, N_DEV)),
            pltpu.SemaphoreType.DMA((HALVES, N_DEV)),
            pltpu.SemaphoreType.DMA((HALVES, N_DEV)),
        ],
        compiler_params=pltpu.CompilerParams(collective_id=0),
    )(x, router_W, route_idx, expert_W, shared_W)
